# baseline (speedup 1.0000x reference)
import functools

import jax
import jax.numpy as jnp
from jax import lax
from jax.experimental import pallas as pl
from jax.experimental.pallas import tpu as pltpu
from jax.experimental.pallas import tpu_sc as plsc

N = 50000
NA = 50048
E = 800000
NC, NS = 2, 16
NW = NC * NS
B = 128
KB = 200
DEPTH = 8
KBP = KB + DEPTH
EPAD = NW * KBP * B
F = 16
RB = 1000
GRID = N // RB
EPS = 1e-5

_MESH = plsc.VectorSubcoreMesh(
    core_axis_name="c", subcore_axis_name="s", num_cores=NC, num_subcores=NS)


def _zero_acc(zeros_v, acc, sid):
    base = sid * 3128
    for r in range(24):
        pltpu.sync_copy(zeros_v, acc.at[pl.ds(base + r * 128, 128)])
    pltpu.sync_copy(zeros_v.at[pl.ds(0, 56)], acc.at[pl.ds(base + 3072, 56)])


def _deg_body(dst_hbm, ones_hbm, z_hbm, out_hbm, dst_v, ones_v, zeros_v, acc):
    cc = lax.axis_index("c")
    sid = lax.axis_index("s")
    wid = cc * NS + sid
    pltpu.sync_copy(dst_hbm.at[wid], dst_v)
    pltpu.sync_copy(ones_hbm, ones_v)
    pltpu.sync_copy(z_hbm, zeros_v)
    _zero_acc(zeros_v, acc, sid)
    plsc.subcore_barrier()

    def body(j, car):
        pltpu.sync_copy(ones_v, acc.at[dst_v.at[j]], add=True)
        return car

    lax.fori_loop(0, KBP, body, 0)
    plsc.subcore_barrier()
    pltpu.sync_copy(acc.at[pl.ds(sid * 3128, 3128)],
                    out_hbm.at[cc, pl.ds(sid * 3128, 3128)])


_SC_PARAMS = pltpu.CompilerParams(use_tc_tiling_on_sc=False)

_deg_kernel = functools.partial(
    pl.kernel,
    out_type=jax.ShapeDtypeStruct((NC, NA, 16), jnp.float32),
    mesh=_MESH,
    compiler_params=_SC_PARAMS,
    scratch_types=[
        pltpu.VMEM((KBP, B), jnp.int32),
        pltpu.VMEM((B, 16), jnp.float32),
        pltpu.VMEM((B, 16), jnp.float32),
        pltpu.VMEM_SHARED((NA, 16), jnp.float32),
    ],
)(_deg_body)


def _make_scatter(C):
    def body(*refs):
        src_hbm, dst_hbm, z_hbm = refs[0], refs[1], refs[2]
        tabs = refs[3:3 + C]
        outs = refs[3 + C:3 + 2 * C]
        src_v, dst_v, zeros_v, buf, acc, gsem, ssem = refs[3 + 2 * C:]
        cc = lax.axis_index("c")
        sid = lax.axis_index("s")
        wid = cc * NS + sid
        pltpu.sync_copy(src_hbm.at[wid], src_v)
        pltpu.sync_copy(dst_hbm.at[wid], dst_v)
        pltpu.sync_copy(z_hbm, zeros_v)
        bsl = [buf.at[pl.ds(k * B, B)] for k in range(DEPTH)]
        for c in range(C):
            _zero_acc(zeros_v, acc, sid)
            plsc.subcore_barrier()
            tab = tabs[c]
            for k in range(DEPTH):
                pltpu.async_copy(tab.at[src_v.at[k]], bsl[k], gsem)

            def grp(q, car):
                j0 = q * DEPTH
                for k in range(DEPTH):
                    pltpu.make_async_copy(
                        tab.at[src_v.at[0]], bsl[k], gsem).wait()
                for k in range(DEPTH):
                    pltpu.async_copy(bsl[k], acc.at[dst_v.at[j0 + k]], ssem,
                                     add=True)
                for k in range(DEPTH):
                    pltpu.make_async_copy(
                        bsl[k], acc.at[dst_v.at[0]], ssem).wait()
                for k in range(DEPTH):
                    pltpu.async_copy(
                        tab.at[src_v.at[j0 + DEPTH + k]], bsl[k], gsem)
                return car

            lax.fori_loop(0, KB // DEPTH, grp, 0)
            for k in range(DEPTH):
                pltpu.make_async_copy(tab.at[src_v.at[0]], bsl[k], gsem).wait()
            plsc.subcore_barrier()
            pltpu.sync_copy(acc.at[pl.ds(sid * 3128, 3128)],
                            outs[c].at[cc, pl.ds(sid * 3128, 3128)])
            plsc.subcore_barrier()

    return pl.kernel(
        body,
        out_type=[jax.ShapeDtypeStruct((NC, NA, F), jnp.float32)] * C,
        mesh=_MESH,
        compiler_params=_SC_PARAMS,
        scratch_types=[
            pltpu.VMEM((KBP, B), jnp.int32),
            pltpu.VMEM((KBP, B), jnp.int32),
            pltpu.VMEM((B, F), jnp.float32),
            pltpu.VMEM((DEPTH * B, F), jnp.float32),
            pltpu.VMEM_SHARED((NA, F), jnp.float32),
            pltpu.SemaphoreType.DMA,
            pltpu.SemaphoreType.DMA,
        ],
    )


_scatter = {C: _make_scatter(C) for C in (2, 4, 8)}


def _row_spec(shape):
    nd = len(shape)
    if nd == 2:
        return pl.BlockSpec((RB, shape[1]), lambda i: (i, 0))
    return pl.BlockSpec((shape[0], RB, shape[2]), lambda i: (0, i, 0))


def _full_spec(shape):
    return pl.BlockSpec(shape, lambda i: (0,) * len(shape))


C1 = 64 // F


def _k0_body(x_ref, dA_ref, dB_ref, W_ref, dinv_ref, *t_refs):
    deg = dA_ref[...][:, 0:1] + dB_ref[...][:, 0:1] + 1.0
    dinv = lax.rsqrt(deg)
    dinv_ref[...] = dinv
    xw = jnp.dot(x_ref[...], W_ref[...],
                 preferred_element_type=jnp.float32) * dinv
    for c in range(C1):
        t_refs[c][...] = xw[:, F * c:F * (c + 1)]


def _k0(x, degp, W1):
    return pl.pallas_call(
        _k0_body,
        grid=(GRID,),
        in_specs=[_row_spec(x.shape),
                  pl.BlockSpec((RB, 16), lambda i: (i, 0)),
                  pl.BlockSpec((RB, 16), lambda i: (i, 0)),
                  _full_spec(W1.shape)],
        out_specs=[_row_spec((N, 1))] + [_row_spec((NA, F))] * C1,
        out_shape=[jax.ShapeDtypeStruct((N, 1), jnp.float32)]
        + [jax.ShapeDtypeStruct((NA, F), jnp.float32)] * C1,
    )(x, degp[0], degp[1], W1)


def _pre_act(dinv_ref, b_ref, p_refs, t_refs):
    parts = [p[...][0] + p[...][1] + t[...] for p, t in zip(p_refs, t_refs)]
    t = parts[0] if len(parts) == 1 else jnp.concatenate(parts, axis=1)
    return t * dinv_ref[...] + b_ref[...]


def _make_stats(C):
    do = F * C

    def body(*refs):
        dinv_ref, b_ref = refs[0], refs[1]
        p_refs = refs[2:2 + C]
        t_refs = refs[2 + C:2 + 2 * C]
        stats_ref, acc_ref = refs[2 + 2 * C], refs[3 + 2 * C]
        i = pl.program_id(0)
        pre = _pre_act(dinv_ref, b_ref, p_refs, t_refs)

        @pl.when(i == 0)
        def _():
            acc_ref[...] = jnp.zeros_like(acc_ref)

        acc_ref[...] += jnp.stack(
            [jnp.sum(pre, axis=0), jnp.sum(pre * pre, axis=0)])

        @pl.when(i == GRID - 1)
        def _():
            stats_ref[...] = acc_ref[...]

    def call(dinv, b, parts, tabs):
        return pl.pallas_call(
            body,
            grid=(GRID,),
            in_specs=[_row_spec((N, 1)), _full_spec((1, do))]
            + [_row_spec((NC, NA, F))] * C + [_row_spec((NA, F))] * C,
            out_specs=_full_spec((2, do)),
            out_shape=jax.ShapeDtypeStruct((2, do), jnp.float32),
            scratch_shapes=[pltpu.VMEM((2, do), jnp.float32)],
        )(dinv, b, *parts, *tabs)

    return call


def _bn_apply(pre, stats_ref, g_ref, be_ref):
    m = stats_ref[...][0:1, :] / N
    v = stats_ref[...][1:2, :] / N - m * m
    rstd = lax.rsqrt(v + EPS)
    return (pre - m) * rstd * g_ref[...] + be_ref[...]


def _make_apply(C, C_next):
    do = F * C

    def body(*refs):
        dinv_ref, b_ref, g_ref, be_ref, stats_ref, W_ref = refs[:6]
        p_refs = refs[6:6 + C]
        t_refs = refs[6 + C:6 + 2 * C]
        o_refs = refs[6 + 2 * C:]
        pre = _pre_act(dinv_ref, b_ref, p_refs, t_refs)
        h = jnp.maximum(_bn_apply(pre, stats_ref, g_ref, be_ref), 0.0)
        xw = jnp.dot(h, W_ref[...],
                     preferred_element_type=jnp.float32) * dinv_ref[...]
        for c2 in range(C_next):
            o_refs[c2][...] = xw[:, F * c2:F * (c2 + 1)]

    def call(dinv, b, g, be, stats, W, parts, tabs):
        return pl.pallas_call(
            body,
            grid=(GRID,),
            in_specs=[_row_spec((N, 1)), _full_spec((1, do)),
                      _full_spec((1, do)), _full_spec((1, do)),
                      _full_spec((2, do)), _full_spec(W.shape)]
            + [_row_spec((NC, NA, F))] * C + [_row_spec((NA, F))] * C,
            out_specs=[_row_spec((NA, F))] * C_next,
            out_shape=[jax.ShapeDtypeStruct((NA, F), jnp.float32)] * C_next,
        )(dinv, b, g, be, stats, W, *parts, *tabs)

    return call


C4 = 32 // F


def _apply4_body(*refs):
    dinv_ref, b_ref, g_ref, be_ref, stats_ref, cW_ref, cb_ref = refs[:7]
    p_refs = refs[7:7 + C4]
    t_refs = refs[7 + C4:7 + 2 * C4]
    d1_ref, cstats_ref, acc_ref = refs[7 + 2 * C4:]
    i = pl.program_id(0)
    pre = _pre_act(dinv_ref, b_ref, p_refs, t_refs)
    h4 = jnp.maximum(_bn_apply(pre, stats_ref, g_ref, be_ref), 0.0)
    d1 = jnp.maximum(
        jnp.dot(h4, cW_ref[...], preferred_element_type=jnp.float32)
        + cb_ref[...], 0.0)
    d1_ref[...] = d1

    @pl.when(i == 0)
    def _():
        acc_ref[...] = jnp.zeros_like(acc_ref)

    acc_ref[...] += jnp.stack([jnp.sum(d1, axis=0), jnp.sum(d1 * d1, axis=0)])

    @pl.when(i == GRID - 1)
    def _():
        cstats_ref[...] = acc_ref[...]


def _apply4(dinv, b, g, be, stats, cW1, cb1, parts, tabs):
    return pl.pallas_call(
        _apply4_body,
        grid=(GRID,),
        in_specs=[_row_spec((N, 1)), _full_spec((1, 32)), _full_spec((1, 32)),
                  _full_spec((1, 32)), _full_spec((2, 32)),
                  _full_spec(cW1.shape), _full_spec((1, 32))]
        + [_row_spec((NC, NA, F))] * C4 + [_row_spec((NA, F))] * C4,
        out_specs=[_row_spec((N, 32)), _full_spec((2, 32))],
        out_shape=[jax.ShapeDtypeStruct((N, 32), jnp.float32),
                   jax.ShapeDtypeStruct((2, 32), jnp.float32)],
        scratch_shapes=[pltpu.VMEM((2, 32), jnp.float32)],
    )(dinv, b, g, be, stats, cW1, cb1, *parts, *tabs)


def _mlp_body(d_ref, stats_ref, g_ref, be_ref, W_ref, wb_ref,
              d2_ref, cstats_ref, acc_ref):
    i = pl.program_id(0)
    e = _bn_apply(d_ref[...], stats_ref, g_ref, be_ref)
    d2 = jnp.maximum(
        jnp.dot(e, W_ref[...], preferred_element_type=jnp.float32)
        + wb_ref[...], 0.0)
    d2_ref[...] = d2

    @pl.when(i == 0)
    def _():
        acc_ref[...] = jnp.zeros_like(acc_ref)

    acc_ref[...] += jnp.stack([jnp.sum(d2, axis=0), jnp.sum(d2 * d2, axis=0)])

    @pl.when(i == GRID - 1)
    def _():
        cstats_ref[...] = acc_ref[...]


def _mlp(d, stats, g, be, W, wb, dn):
    di = d.shape[1]
    return pl.pallas_call(
        _mlp_body,
        grid=(GRID,),
        in_specs=[_row_spec((N, di)), _full_spec((2, di)),
                  _full_spec((1, di)), _full_spec((1, di)),
                  _full_spec(W.shape), _full_spec((1, dn))],
        out_specs=[_row_spec((N, dn)), _full_spec((2, dn))],
        out_shape=[jax.ShapeDtypeStruct((N, dn), jnp.float32),
                   jax.ShapeDtypeStruct((2, dn), jnp.float32)],
        scratch_shapes=[pltpu.VMEM((2, dn), jnp.float32)],
    )(d, stats, g, be, W, wb)


def _final_body(d_ref, stats_ref, g_ref, be_ref, W_ref, wb_ref, out_ref):
    e = _bn_apply(d_ref[...], stats_ref, g_ref, be_ref)
    out_ref[...] = (jnp.dot(e, W_ref[...], preferred_element_type=jnp.float32)
                    + wb_ref[...])


def _final(d, stats, g, be, W, wb):
    di, dn = W.shape
    return pl.pallas_call(
        _final_body,
        grid=(GRID,),
        in_specs=[_row_spec((N, di)), _full_spec((2, di)),
                  _full_spec((1, di)), _full_spec((1, di)),
                  _full_spec(W.shape), _full_spec((1, dn))],
        out_specs=_row_spec((N, dn)),
        out_shape=jax.ShapeDtypeStruct((N, dn), jnp.float32),
    )(d, stats, g, be, W, wb)


def kernel(x, edge_index, W1, b1, g1, be1, W2, b2, g2, be2, W3, b3, g3, be3,
           W4, b4, g4, be4, cW1, cb1, cW2, cb2, cW3, cb3, cg1, cbe1,
           cg2, cbe2):
    pad = jnp.full((NW * KB * B - E,), N, jnp.int32)
    tail = jnp.full((NW, DEPTH, B), N, jnp.int32)

    def tile_idx(row):
        main = jnp.concatenate([row, pad]).reshape(NW, KB, B)
        return jnp.concatenate([main, tail], axis=1)

    src_t = tile_idx(edge_index[0])
    dst_t = tile_idx(edge_index[1])
    ones16 = jnp.ones((B, 16), jnp.float32)
    z16 = jnp.zeros((B, 16), jnp.float32)
    zF = jnp.zeros((B, F), jnp.float32)
    r2 = lambda v: v.reshape(1, -1)

    degp = _deg_kernel(dst_t, ones16, z16)
    k0_out = _k0(x, degp, W1)
    dinv, tabs = k0_out[0], list(k0_out[1:])

    layer_params = [(b1, g1, be1, 4, W2, 8), (b2, g2, be2, 8, W3, 4),
                    (b3, g3, be3, 4, W4, 2)]
    for b_, g_, be_, C, Wn, Cn in layer_params:
        parts = _scatter[C](src_t, dst_t, zF, *tabs)
        parts = list(parts) if isinstance(parts, (list, tuple)) else [parts]
        stats = _make_stats(C)(dinv, r2(b_), parts, tabs)
        tabs = _make_apply(C, Cn)(dinv, r2(b_), r2(g_), r2(be_), stats, Wn,
                                  parts, tabs)
        tabs = list(tabs) if isinstance(tabs, (list, tuple)) else [tabs]

    parts = _scatter[2](src_t, dst_t, zF, *tabs)
    parts = list(parts) if isinstance(parts, (list, tuple)) else [parts]
    stats4 = _make_stats(2)(dinv, r2(b4), parts, tabs)
    d1, cs1 = _apply4(dinv, r2(b4), r2(g4), r2(be4), stats4, cW1, r2(cb1),
                      parts, tabs)
    d2, cs2 = _mlp(d1, cs1, r2(cg1), r2(cbe1), cW2, r2(cb2), 16)
    out = _final(d2, cs2, r2(cg2), r2(cbe2), cW3, r2(cb3))
    return out


# speedup vs baseline: 1.7016x; 1.7016x over previous
import functools

import jax
import jax.numpy as jnp
from jax import lax
from jax.experimental import pallas as pl
from jax.experimental.pallas import tpu as pltpu
from jax.experimental.pallas import tpu_sc as plsc

N = 50000
NA = 50048
E = 800000
NC, NS = 2, 16
NW = NC * NS
B = 128
KB = 200
DEPTH = 8
KBP = KB + DEPTH
EPAD = NW * KBP * B
F = 16
RB = 1000
GRID = N // RB
EPS = 1e-5

_MESH = plsc.VectorSubcoreMesh(
    core_axis_name="c", subcore_axis_name="s", num_cores=NC, num_subcores=NS)


def _zero_acc(zeros_v, acc, sid):
    base = sid * 3128
    for r in range(24):
        pltpu.sync_copy(zeros_v, acc.at[pl.ds(base + r * 128, 128)])
    pltpu.sync_copy(zeros_v.at[pl.ds(0, 56)], acc.at[pl.ds(base + 3072, 56)])


def _deg_body(dst_hbm, ones_hbm, z_hbm, out_hbm, dst_v, ones_v, zeros_v, acc):
    cc = lax.axis_index("c")
    sid = lax.axis_index("s")
    wid = cc * NS + sid
    pltpu.sync_copy(dst_hbm.at[wid], dst_v)
    pltpu.sync_copy(ones_hbm, ones_v)
    pltpu.sync_copy(z_hbm, zeros_v)
    _zero_acc(zeros_v, acc, sid)
    plsc.subcore_barrier()

    def body(j, car):
        pltpu.sync_copy(ones_v, acc.at[dst_v.at[j]], add=True)
        return car

    lax.fori_loop(0, KBP, body, 0)
    plsc.subcore_barrier()
    pltpu.sync_copy(acc.at[pl.ds(sid * 3128, 3128)],
                    out_hbm.at[cc, pl.ds(sid * 3128, 3128)])


_SC_PARAMS = pltpu.CompilerParams(use_tc_tiling_on_sc=False)

_deg_kernel = functools.partial(
    pl.kernel,
    out_type=jax.ShapeDtypeStruct((NC, NA, 16), jnp.float32),
    mesh=_MESH,
    compiler_params=_SC_PARAMS,
    scratch_types=[
        pltpu.VMEM((KBP, B), jnp.int32),
        pltpu.VMEM((B, 16), jnp.float32),
        pltpu.VMEM((B, 16), jnp.float32),
        pltpu.VMEM_SHARED((NA, 16), jnp.float32),
    ],
)(_deg_body)


def _make_scatter(C):
    def body(*refs):
        src_hbm, dst_hbm, z_hbm = refs[0], refs[1], refs[2]
        tabs = refs[3:3 + C]
        outs = refs[3 + C:3 + 2 * C]
        src8, dst8, zeros_v, buf, tab_s, acc = refs[3 + 2 * C:]
        cc = lax.axis_index("c")
        sid = lax.axis_index("s")
        wid = cc * NS + sid
        pltpu.sync_copy(z_hbm, zeros_v)
        sl = pl.ds(sid * 3128, 3128)
        for c in range(C):
            pltpu.sync_copy(tabs[c].at[sl], tab_s.at[sl])
            _zero_acc(zeros_v, acc, sid)
            plsc.subcore_barrier()

            def grp(q, car):
                pltpu.sync_copy(src_hbm.at[wid, pl.ds(q * DEPTH, DEPTH)], src8)
                pltpu.sync_copy(dst_hbm.at[wid, pl.ds(q * DEPTH, DEPTH)], dst8)
                for k in range(DEPTH):
                    pltpu.sync_copy(tab_s.at[src8.at[k]], buf)
                    pltpu.sync_copy(buf, acc.at[dst8.at[k]], add=True)
                return car

            lax.fori_loop(0, KB // DEPTH, grp, 0)
            plsc.subcore_barrier()
            pltpu.sync_copy(acc.at[sl], outs[c].at[cc, sl])
            plsc.subcore_barrier()

    return pl.kernel(
        body,
        out_type=[jax.ShapeDtypeStruct((NC, NA, F), jnp.float32)] * C,
        mesh=_MESH,
        compiler_params=_SC_PARAMS,
        scratch_types=[
            pltpu.VMEM((DEPTH, B), jnp.int32),
            pltpu.VMEM((DEPTH, B), jnp.int32),
            pltpu.VMEM((B, F), jnp.float32),
            pltpu.VMEM((B, F), jnp.float32),
            pltpu.VMEM_SHARED((NA, F), jnp.float32),
            pltpu.VMEM_SHARED((NA, F), jnp.float32),
        ],
    )


_scatter = {C: _make_scatter(C) for C in (2, 4, 8)}


def _row_spec(shape):
    nd = len(shape)
    if nd == 2:
        return pl.BlockSpec((RB, shape[1]), lambda i: (i, 0))
    return pl.BlockSpec((shape[0], RB, shape[2]), lambda i: (0, i, 0))


def _full_spec(shape):
    return pl.BlockSpec(shape, lambda i: (0,) * len(shape))


C1 = 64 // F


def _k0_body(x_ref, dA_ref, dB_ref, W_ref, dinv_ref, *t_refs):
    deg = dA_ref[...][:, 0:1] + dB_ref[...][:, 0:1] + 1.0
    dinv = lax.rsqrt(deg)
    dinv_ref[...] = dinv
    xw = jnp.dot(x_ref[...], W_ref[...],
                 preferred_element_type=jnp.float32) * dinv
    for c in range(C1):
        t_refs[c][...] = xw[:, F * c:F * (c + 1)]


def _k0(x, degp, W1):
    return pl.pallas_call(
        _k0_body,
        grid=(GRID,),
        in_specs=[_row_spec(x.shape),
                  pl.BlockSpec((RB, 16), lambda i: (i, 0)),
                  pl.BlockSpec((RB, 16), lambda i: (i, 0)),
                  _full_spec(W1.shape)],
        out_specs=[_row_spec((N, 1))] + [_row_spec((NA, F))] * C1,
        out_shape=[jax.ShapeDtypeStruct((N, 1), jnp.float32)]
        + [jax.ShapeDtypeStruct((NA, F), jnp.float32)] * C1,
    )(x, degp[0], degp[1], W1)


def _pre_act(dinv_ref, b_ref, p_refs, t_refs):
    parts = [p[...][0] + p[...][1] + t[...] for p, t in zip(p_refs, t_refs)]
    t = parts[0] if len(parts) == 1 else jnp.concatenate(parts, axis=1)
    return t * dinv_ref[...] + b_ref[...]


def _make_stats(C):
    do = F * C

    def body(*refs):
        dinv_ref, b_ref = refs[0], refs[1]
        p_refs = refs[2:2 + C]
        t_refs = refs[2 + C:2 + 2 * C]
        stats_ref, acc_ref = refs[2 + 2 * C], refs[3 + 2 * C]
        i = pl.program_id(0)
        pre = _pre_act(dinv_ref, b_ref, p_refs, t_refs)

        @pl.when(i == 0)
        def _():
            acc_ref[...] = jnp.zeros_like(acc_ref)

        acc_ref[...] += jnp.stack(
            [jnp.sum(pre, axis=0), jnp.sum(pre * pre, axis=0)])

        @pl.when(i == GRID - 1)
        def _():
            stats_ref[...] = acc_ref[...]

    def call(dinv, b, parts, tabs):
        return pl.pallas_call(
            body,
            grid=(GRID,),
            in_specs=[_row_spec((N, 1)), _full_spec((1, do))]
            + [_row_spec((NC, NA, F))] * C + [_row_spec((NA, F))] * C,
            out_specs=_full_spec((2, do)),
            out_shape=jax.ShapeDtypeStruct((2, do), jnp.float32),
            scratch_shapes=[pltpu.VMEM((2, do), jnp.float32)],
        )(dinv, b, *parts, *tabs)

    return call


def _bn_apply(pre, stats_ref, g_ref, be_ref):
    m = stats_ref[...][0:1, :] / N
    v = stats_ref[...][1:2, :] / N - m * m
    rstd = lax.rsqrt(v + EPS)
    return (pre - m) * rstd * g_ref[...] + be_ref[...]


def _make_apply(C, C_next):
    do = F * C

    def body(*refs):
        dinv_ref, b_ref, g_ref, be_ref, stats_ref, W_ref = refs[:6]
        p_refs = refs[6:6 + C]
        t_refs = refs[6 + C:6 + 2 * C]
        o_refs = refs[6 + 2 * C:]
        pre = _pre_act(dinv_ref, b_ref, p_refs, t_refs)
        h = jnp.maximum(_bn_apply(pre, stats_ref, g_ref, be_ref), 0.0)
        xw = jnp.dot(h, W_ref[...],
                     preferred_element_type=jnp.float32) * dinv_ref[...]
        for c2 in range(C_next):
            o_refs[c2][...] = xw[:, F * c2:F * (c2 + 1)]

    def call(dinv, b, g, be, stats, W, parts, tabs):
        return pl.pallas_call(
            body,
            grid=(GRID,),
            in_specs=[_row_spec((N, 1)), _full_spec((1, do)),
                      _full_spec((1, do)), _full_spec((1, do)),
                      _full_spec((2, do)), _full_spec(W.shape)]
            + [_row_spec((NC, NA, F))] * C + [_row_spec((NA, F))] * C,
            out_specs=[_row_spec((NA, F))] * C_next,
            out_shape=[jax.ShapeDtypeStruct((NA, F), jnp.float32)] * C_next,
        )(dinv, b, g, be, stats, W, *parts, *tabs)

    return call


C4 = 32 // F


def _apply4_body(*refs):
    dinv_ref, b_ref, g_ref, be_ref, stats_ref, cW_ref, cb_ref = refs[:7]
    p_refs = refs[7:7 + C4]
    t_refs = refs[7 + C4:7 + 2 * C4]
    d1_ref, cstats_ref, acc_ref = refs[7 + 2 * C4:]
    i = pl.program_id(0)
    pre = _pre_act(dinv_ref, b_ref, p_refs, t_refs)
    h4 = jnp.maximum(_bn_apply(pre, stats_ref, g_ref, be_ref), 0.0)
    d1 = jnp.maximum(
        jnp.dot(h4, cW_ref[...], preferred_element_type=jnp.float32)
        + cb_ref[...], 0.0)
    d1_ref[...] = d1

    @pl.when(i == 0)
    def _():
        acc_ref[...] = jnp.zeros_like(acc_ref)

    acc_ref[...] += jnp.stack([jnp.sum(d1, axis=0), jnp.sum(d1 * d1, axis=0)])

    @pl.when(i == GRID - 1)
    def _():
        cstats_ref[...] = acc_ref[...]


def _apply4(dinv, b, g, be, stats, cW1, cb1, parts, tabs):
    return pl.pallas_call(
        _apply4_body,
        grid=(GRID,),
        in_specs=[_row_spec((N, 1)), _full_spec((1, 32)), _full_spec((1, 32)),
                  _full_spec((1, 32)), _full_spec((2, 32)),
                  _full_spec(cW1.shape), _full_spec((1, 32))]
        + [_row_spec((NC, NA, F))] * C4 + [_row_spec((NA, F))] * C4,
        out_specs=[_row_spec((N, 32)), _full_spec((2, 32))],
        out_shape=[jax.ShapeDtypeStruct((N, 32), jnp.float32),
                   jax.ShapeDtypeStruct((2, 32), jnp.float32)],
        scratch_shapes=[pltpu.VMEM((2, 32), jnp.float32)],
    )(dinv, b, g, be, stats, cW1, cb1, *parts, *tabs)


def _mlp_body(d_ref, stats_ref, g_ref, be_ref, W_ref, wb_ref,
              d2_ref, cstats_ref, acc_ref):
    i = pl.program_id(0)
    e = _bn_apply(d_ref[...], stats_ref, g_ref, be_ref)
    d2 = jnp.maximum(
        jnp.dot(e, W_ref[...], preferred_element_type=jnp.float32)
        + wb_ref[...], 0.0)
    d2_ref[...] = d2

    @pl.when(i == 0)
    def _():
        acc_ref[...] = jnp.zeros_like(acc_ref)

    acc_ref[...] += jnp.stack([jnp.sum(d2, axis=0), jnp.sum(d2 * d2, axis=0)])

    @pl.when(i == GRID - 1)
    def _():
        cstats_ref[...] = acc_ref[...]


def _mlp(d, stats, g, be, W, wb, dn):
    di = d.shape[1]
    return pl.pallas_call(
        _mlp_body,
        grid=(GRID,),
        in_specs=[_row_spec((N, di)), _full_spec((2, di)),
                  _full_spec((1, di)), _full_spec((1, di)),
                  _full_spec(W.shape), _full_spec((1, dn))],
        out_specs=[_row_spec((N, dn)), _full_spec((2, dn))],
        out_shape=[jax.ShapeDtypeStruct((N, dn), jnp.float32),
                   jax.ShapeDtypeStruct((2, dn), jnp.float32)],
        scratch_shapes=[pltpu.VMEM((2, dn), jnp.float32)],
    )(d, stats, g, be, W, wb)


def _final_body(d_ref, stats_ref, g_ref, be_ref, W_ref, wb_ref, out_ref):
    e = _bn_apply(d_ref[...], stats_ref, g_ref, be_ref)
    out_ref[...] = (jnp.dot(e, W_ref[...], preferred_element_type=jnp.float32)
                    + wb_ref[...])


def _final(d, stats, g, be, W, wb):
    di, dn = W.shape
    return pl.pallas_call(
        _final_body,
        grid=(GRID,),
        in_specs=[_row_spec((N, di)), _full_spec((2, di)),
                  _full_spec((1, di)), _full_spec((1, di)),
                  _full_spec(W.shape), _full_spec((1, dn))],
        out_specs=_row_spec((N, dn)),
        out_shape=jax.ShapeDtypeStruct((N, dn), jnp.float32),
    )(d, stats, g, be, W, wb)


def kernel(x, edge_index, W1, b1, g1, be1, W2, b2, g2, be2, W3, b3, g3, be3,
           W4, b4, g4, be4, cW1, cb1, cW2, cb2, cW3, cb3, cg1, cbe1,
           cg2, cbe2):
    pad = jnp.full((NW * KB * B - E,), N, jnp.int32)
    tail = jnp.full((NW, DEPTH, B), N, jnp.int32)

    def tile_idx(row):
        main = jnp.concatenate([row, pad]).reshape(NW, KB, B)
        return jnp.concatenate([main, tail], axis=1)

    src_t = tile_idx(edge_index[0])
    dst_t = tile_idx(edge_index[1])
    ones16 = jnp.ones((B, 16), jnp.float32)
    z16 = jnp.zeros((B, 16), jnp.float32)
    zF = jnp.zeros((B, F), jnp.float32)
    r2 = lambda v: v.reshape(1, -1)

    degp = _deg_kernel(dst_t, ones16, z16)
    k0_out = _k0(x, degp, W1)
    dinv, tabs = k0_out[0], list(k0_out[1:])

    layer_params = [(b1, g1, be1, 4, W2, 8), (b2, g2, be2, 8, W3, 4),
                    (b3, g3, be3, 4, W4, 2)]
    for b_, g_, be_, C, Wn, Cn in layer_params:
        parts = _scatter[C](src_t, dst_t, zF, *tabs)
        parts = list(parts) if isinstance(parts, (list, tuple)) else [parts]
        stats = _make_stats(C)(dinv, r2(b_), parts, tabs)
        tabs = _make_apply(C, Cn)(dinv, r2(b_), r2(g_), r2(be_), stats, Wn,
                                  parts, tabs)
        tabs = list(tabs) if isinstance(tabs, (list, tuple)) else [tabs]

    parts = _scatter[2](src_t, dst_t, zF, *tabs)
    parts = list(parts) if isinstance(parts, (list, tuple)) else [parts]
    stats4 = _make_stats(2)(dinv, r2(b4), parts, tabs)
    d1, cs1 = _apply4(dinv, r2(b4), r2(g4), r2(be4), stats4, cW1, r2(cb1),
                      parts, tabs)
    d2, cs2 = _mlp(d1, cs1, r2(cg1), r2(cbe1), cW2, r2(cb2), 16)
    out = _final(d2, cs2, r2(cg2), r2(cbe2), cW3, r2(cb3))
    return out


# speedup vs baseline: 3.6844x; 2.1652x over previous
import functools

import jax
import jax.numpy as jnp
from jax import lax
from jax.experimental import pallas as pl
from jax.experimental.pallas import tpu as pltpu
from jax.experimental.pallas import tpu_sc as plsc

N = 50000
NA = 51200
E = 800000
NC, NS = 2, 16
NW = NC * NS
B = 128
KB = 200
DEPTH = 8
KBP = KB + DEPTH
EPAD = NW * KBP * B
F = 16
RB = 1000
GRID = N // RB
EPS = 1e-5

_MESH = plsc.VectorSubcoreMesh(
    core_axis_name="c", subcore_axis_name="s", num_cores=NC, num_subcores=NS)


def _zero_acc(zeros_v, acc, sid):
    base = sid * 3200
    for r in range(25):
        pltpu.sync_copy(zeros_v, acc.at[pl.ds(base + r * 128, 128)])


def _deg_body(dst_hbm, ones_hbm, z_hbm, out_hbm, dst_v, ones_v, zeros_v, acc):
    cc = lax.axis_index("c")
    sid = lax.axis_index("s")
    wid = cc * NS + sid
    pltpu.sync_copy(dst_hbm.at[wid], dst_v)
    pltpu.sync_copy(ones_hbm, ones_v)
    pltpu.sync_copy(z_hbm, zeros_v)
    _zero_acc(zeros_v, acc, sid)
    plsc.subcore_barrier()

    def body(j, car):
        pltpu.sync_copy(ones_v, acc.at[dst_v.at[j]], add=True)
        return car

    lax.fori_loop(0, KBP, body, 0)
    plsc.subcore_barrier()
    pltpu.sync_copy(acc.at[pl.ds(sid * 3200, 3200)],
                    out_hbm.at[cc, pl.ds(sid * 3200, 3200)])


_SC_PARAMS = pltpu.CompilerParams(use_tc_tiling_on_sc=False)

_deg_kernel = functools.partial(
    pl.kernel,
    out_type=jax.ShapeDtypeStruct((NC, NA, 16), jnp.float32),
    mesh=_MESH,
    compiler_params=_SC_PARAMS,
    scratch_types=[
        pltpu.VMEM((KBP, B), jnp.int32),
        pltpu.VMEM((B, 16), jnp.float32),
        pltpu.VMEM((B, 16), jnp.float32),
        pltpu.VMEM_SHARED((NA, 16), jnp.float32),
    ],
)(_deg_body)


def _make_scatter(C):
    def body(*refs):
        src_hbm, dst_hbm, z_hbm = refs[0], refs[1], refs[2]
        tabs = refs[3:3 + C]
        outs = refs[3 + C:3 + 2 * C]
        src8, dst8, zeros_v, buf, tab_s, acc, gsem = refs[3 + 2 * C:]
        cc = lax.axis_index("c")
        sid = lax.axis_index("s")
        wid = cc * NS + sid
        pltpu.sync_copy(z_hbm, zeros_v)
        sl = pl.ds(sid * 3200, 3200)
        bufs = [buf.at[pl.ds(0, B)], buf.at[pl.ds(B, B)]]
        slabs = [(0, 88), (88, 88), (176, 24)]
        for c in range(C):
            pltpu.sync_copy(tabs[c].at[sl], tab_s.at[sl])
            _zero_acc(zeros_v, acc, sid)
            plsc.subcore_barrier()
            for off, n in slabs:
                pltpu.sync_copy(src_hbm.at[wid, pl.ds(off, n)],
                                src8.at[pl.ds(0, n)])
                pltpu.sync_copy(dst_hbm.at[wid, pl.ds(off, n)],
                                dst8.at[pl.ds(0, n)])
                pltpu.async_copy(tab_s.at[src8.at[0]], bufs[0], gsem)

                def bat(q, car):
                    pltpu.make_async_copy(
                        tab_s.at[src8.at[0]], bufs[0], gsem).wait()
                    pltpu.async_copy(tab_s.at[src8.at[q + 1]], bufs[1], gsem)
                    pltpu.sync_copy(bufs[0], acc.at[dst8.at[q]], add=True)
                    pltpu.make_async_copy(
                        tab_s.at[src8.at[0]], bufs[1], gsem).wait()
                    pltpu.async_copy(tab_s.at[src8.at[q + 2]], bufs[0], gsem)
                    pltpu.sync_copy(bufs[1], acc.at[dst8.at[q + 1]], add=True)
                    return car

                lax.fori_loop(0, (n - 2) // 2, lambda q, car: bat(2 * q, car),
                              0)
                pltpu.make_async_copy(
                    tab_s.at[src8.at[0]], bufs[0], gsem).wait()
                pltpu.async_copy(tab_s.at[src8.at[n - 1]], bufs[1], gsem)
                pltpu.sync_copy(bufs[0], acc.at[dst8.at[n - 2]], add=True)
                pltpu.make_async_copy(
                    tab_s.at[src8.at[0]], bufs[1], gsem).wait()
                pltpu.sync_copy(bufs[1], acc.at[dst8.at[n - 1]], add=True)
            plsc.subcore_barrier()
            pltpu.sync_copy(acc.at[sl], outs[c].at[cc, sl])
            plsc.subcore_barrier()

    return pl.kernel(
        body,
        out_type=[jax.ShapeDtypeStruct((NC, NA, F), jnp.float32)] * C,
        mesh=_MESH,
        compiler_params=_SC_PARAMS,
        scratch_types=[
            pltpu.VMEM((88, B), jnp.int32),
            pltpu.VMEM((88, B), jnp.int32),
            pltpu.VMEM((B, F), jnp.float32),
            pltpu.VMEM((2 * B, F), jnp.float32),
            pltpu.VMEM_SHARED((NA, F), jnp.float32),
            pltpu.VMEM_SHARED((NA, F), jnp.float32),
            pltpu.SemaphoreType.DMA,
        ],
    )


_scatter = {C: _make_scatter(C) for C in (2, 4, 8)}


def _row_spec(shape):
    nd = len(shape)
    if nd == 2:
        return pl.BlockSpec((RB, shape[1]), lambda i: (i, 0))
    return pl.BlockSpec((shape[0], RB, shape[2]), lambda i: (0, i, 0))


def _full_spec(shape):
    return pl.BlockSpec(shape, lambda i: (0,) * len(shape))


C1 = 64 // F
RBP = 1024
PR = RBP // 8
NP8 = NA // 8
VROWS = N // 8


def _prow_spec(nd3=False, lanes=128):
    if nd3:
        return pl.BlockSpec((NC, PR, lanes), lambda i: (0, i, 0))
    return pl.BlockSpec((PR, lanes), lambda i: (i, 0))


def _full_spec(shape):
    return pl.BlockSpec(shape, lambda i: (0,) * len(shape))


def _row_spec(shape):
    return pl.BlockSpec((RB, shape[1]), lambda i: (i, 0))


def _bcast16(x):
    segs = [jnp.broadcast_to(x[:, k * 16:k * 16 + 1], (PR, 16))
            for k in range(8)]
    return jnp.concatenate(segs, axis=1)


def _fold16(s):
    return sum(s[:, k * 16:(k + 1) * 16] for k in range(8))


def _rowmask(i, x):
    gr = i * PR + lax.broadcasted_iota(jnp.int32, (PR, 1), 0)
    return jnp.where(gr < VROWS, x, 0.0)


def _k0_body(x_ref, dA_ref, W_refs_and_outs):
    pass


def _k0_body_impl(x_ref, dA_ref, Wb_refs, dinv_ref, t_refs):
    dsum = dA_ref[...][0] + dA_ref[...][1]
    deg = _bcast16(dsum) + 1.0
    dinv_p = lax.rsqrt(deg)
    dinv_ref[...] = dinv_p
    xp = x_ref[...]
    for c in range(C1):
        t_refs[c][...] = jnp.dot(
            xp, Wb_refs[c][...],
            preferred_element_type=jnp.float32) * dinv_p


def _k0(xp, degp, W1bs):
    def body(*refs):
        x_ref, dA_ref = refs[0], refs[1]
        Wb_refs = refs[2:2 + C1]
        dinv_ref = refs[2 + C1]
        t_refs = refs[3 + C1:]
        _k0_body_impl(x_ref, dA_ref, Wb_refs, dinv_ref, t_refs)

    return pl.pallas_call(
        body,
        grid=(GRID,),
        in_specs=[_prow_spec(lanes=336), _prow_spec(True)]
        + [_full_spec((336, 128))] * C1,
        out_specs=[_prow_spec()] * (1 + C1),
        out_shape=[jax.ShapeDtypeStruct((NP8, 128), jnp.float32)] * (1 + C1),
    )(xp, degp, *W1bs)


def _pre_chunks(dinv_p, b_ref, p_refs, t_refs):
    pres = []
    for c, (p, t) in enumerate(zip(p_refs, t_refs)):
        s = p[...][0] + p[...][1] + t[...]
        pres.append(s * dinv_p + b_ref[...][:, c * 128:(c + 1) * 128])
    return pres


def _make_stats(C):
    def body(*refs):
        dinv_ref, b_ref = refs[0], refs[1]
        p_refs = refs[2:2 + C]
        t_refs = refs[2 + C:2 + 2 * C]
        stats_ref, acc_ref = refs[2 + 2 * C], refs[3 + 2 * C]
        i = pl.program_id(0)
        pres = _pre_chunks(dinv_ref[...], b_ref, p_refs, t_refs)
        pre = pres[0] if C == 1 else jnp.concatenate(pres, axis=1)
        pre = _rowmask(i, pre)

        @pl.when(i == 0)
        def _():
            acc_ref[...] = jnp.zeros_like(acc_ref)

        acc_ref[...] += jnp.stack(
            [jnp.sum(pre, axis=0), jnp.sum(pre * pre, axis=0)])

        @pl.when(i == GRID - 1)
        def _():
            stats_ref[...] = acc_ref[...]

    def call(dinv, b, parts, tabs):
        C_ = len(tabs)
        return pl.pallas_call(
            body,
            grid=(GRID,),
            in_specs=[_prow_spec(), _full_spec((1, C_ * 128))]
            + [_prow_spec(True)] * C_ + [_prow_spec()] * C_,
            out_specs=_full_spec((2, C_ * 128)),
            out_shape=jax.ShapeDtypeStruct((2, C_ * 128), jnp.float32),
            scratch_shapes=[pltpu.VMEM((2, C_ * 128), jnp.float32)],
        )(dinv, b, *parts, *tabs)

    return call


def _bn_packed(pre_c, stats_ref, g_ref, be_ref, c):
    s = _fold16(stats_ref[...][:, c * 128:(c + 1) * 128])
    m16 = s[0:1, :] / N
    v16 = s[1:2, :] / N - m16 * m16
    r16 = lax.rsqrt(v16 + EPS)
    m_p = jnp.concatenate([m16] * 8, axis=1)
    r_p = jnp.concatenate([r16] * 8, axis=1)
    gseg = g_ref[...][:, c * 128:(c + 1) * 128]
    beseg = be_ref[...][:, c * 128:(c + 1) * 128]
    return (pre_c - m_p) * r_p * gseg + beseg


def _make_apply(C, C_next):
    def body(*refs):
        dinv_ref, b_ref, g_ref, be_ref, stats_ref = refs[:5]
        Wb_refs = refs[5:5 + C * C_next]
        p_refs = refs[5 + C * C_next:5 + C * C_next + C]
        t_refs = refs[5 + C * C_next + C:5 + C * C_next + 2 * C]
        o_refs = refs[5 + C * C_next + 2 * C:]
        dinv_p = dinv_ref[...]
        pres = _pre_chunks(dinv_p, b_ref, p_refs, t_refs)
        hs = [jnp.maximum(_bn_packed(pres[c], stats_ref, g_ref, be_ref, c),
                          0.0) for c in range(C)]
        for c2 in range(C_next):
            xw = hs[0] @ Wb_refs[c2][...]
            for c in range(1, C):
                xw += hs[c] @ Wb_refs[c * C_next + c2][...]
            o_refs[c2][...] = xw * dinv_p

    def call(dinv, b, g, be, stats, Wbs, parts, tabs):
        return pl.pallas_call(
            body,
            grid=(GRID,),
            in_specs=[_prow_spec(), _full_spec((1, C * 128)),
                      _full_spec((1, C * 128)), _full_spec((1, C * 128)),
                      _full_spec((2, C * 128))]
            + [_full_spec((128, 128))] * (C * C_next)
            + [_prow_spec(True)] * C + [_prow_spec()] * C,
            out_specs=[_prow_spec()] * C_next,
            out_shape=[jax.ShapeDtypeStruct((NP8, 128), jnp.float32)]
            * C_next,
        )(dinv, b, g, be, stats, *Wbs, *parts, *tabs)

    return call


C4 = 32 // F


def _apply4_body(*refs):
    dinv_ref, b_ref, g_ref, be_ref, stats_ref, cb_ref = refs[:6]
    Wb_refs = refs[6:6 + C4 * C4]
    p_refs = refs[6 + C4 * C4:6 + C4 * C4 + C4]
    t_refs = refs[6 + C4 * C4 + C4:6 + C4 * C4 + 2 * C4]
    d1_refs = refs[6 + C4 * C4 + 2 * C4:6 + C4 * C4 + 3 * C4]
    cstats_ref, acc_ref = refs[6 + C4 * C4 + 3 * C4:]
    i = pl.program_id(0)
    dinv_p = dinv_ref[...]
    pres = _pre_chunks(dinv_p, b_ref, p_refs, t_refs)
    hs = [jnp.maximum(_bn_packed(pres[c], stats_ref, g_ref, be_ref, c), 0.0)
          for c in range(C4)]
    d1m = []
    for c2 in range(C4):
        xw = hs[0] @ Wb_refs[c2][...]
        for c in range(1, C4):
            xw += hs[c] @ Wb_refs[c * C4 + c2][...]
        d1 = jnp.maximum(xw + cb_ref[...][:, c2 * 128:(c2 + 1) * 128], 0.0)
        d1_refs[c2][...] = d1
        d1m.append(_rowmask(i, d1))
    d1cat = jnp.concatenate(d1m, axis=1)

    @pl.when(i == 0)
    def _():
        acc_ref[...] = jnp.zeros_like(acc_ref)

    acc_ref[...] += jnp.stack(
        [jnp.sum(d1cat, axis=0), jnp.sum(d1cat * d1cat, axis=0)])

    @pl.when(i == GRID - 1)
    def _():
        cstats_ref[...] = acc_ref[...]


def _apply4(dinv, b, g, be, stats, cWbs, cb, parts, tabs):
    return pl.pallas_call(
        _apply4_body,
        grid=(GRID,),
        in_specs=[_prow_spec(), _full_spec((1, C4 * 128)),
                  _full_spec((1, C4 * 128)), _full_spec((1, C4 * 128)),
                  _full_spec((2, C4 * 128)), _full_spec((1, C4 * 128))]
        + [_full_spec((128, 128))] * (C4 * C4)
        + [_prow_spec(True)] * C4 + [_prow_spec()] * C4,
        out_specs=[_prow_spec()] * C4 + [_full_spec((2, C4 * 128))],
        out_shape=[jax.ShapeDtypeStruct((NP8, 128), jnp.float32)] * C4
        + [jax.ShapeDtypeStruct((2, C4 * 128), jnp.float32)],
        scratch_shapes=[pltpu.VMEM((2, C4 * 128), jnp.float32)],
    )(dinv, b, g, be, stats, cb, *cWbs, *parts, *tabs)

def _bn_apply(pre, stats_ref, g_ref, be_ref):
    m = stats_ref[...][0:1, :] / N
    v = stats_ref[...][1:2, :] / N - m * m
    rstd = lax.rsqrt(v + EPS)
    return (pre - m) * rstd * g_ref[...] + be_ref[...]


def _mlp_body(d_ref, stats_ref, g_ref, be_ref, W_ref, wb_ref,
              d2_ref, cstats_ref, acc_ref):
    i = pl.program_id(0)
    e = _bn_apply(d_ref[...], stats_ref, g_ref, be_ref)
    d2 = jnp.maximum(
        jnp.dot(e, W_ref[...], preferred_element_type=jnp.float32)
        + wb_ref[...], 0.0)
    d2_ref[...] = d2

    @pl.when(i == 0)
    def _():
        acc_ref[...] = jnp.zeros_like(acc_ref)

    acc_ref[...] += jnp.stack([jnp.sum(d2, axis=0), jnp.sum(d2 * d2, axis=0)])

    @pl.when(i == GRID - 1)
    def _():
        cstats_ref[...] = acc_ref[...]


def _mlp(d, stats, g, be, W, wb, dn):
    di = d.shape[1]
    return pl.pallas_call(
        _mlp_body,
        grid=(GRID,),
        in_specs=[_row_spec((N, di)), _full_spec((2, di)),
                  _full_spec((1, di)), _full_spec((1, di)),
                  _full_spec(W.shape), _full_spec((1, dn))],
        out_specs=[_row_spec((N, dn)), _full_spec((2, dn))],
        out_shape=[jax.ShapeDtypeStruct((N, dn), jnp.float32),
                   jax.ShapeDtypeStruct((2, dn), jnp.float32)],
        scratch_shapes=[pltpu.VMEM((2, dn), jnp.float32)],
    )(d, stats, g, be, W, wb)


def _final_body(d_ref, stats_ref, g_ref, be_ref, W_ref, wb_ref, out_ref):
    e = _bn_apply(d_ref[...], stats_ref, g_ref, be_ref)
    out_ref[...] = (jnp.dot(e, W_ref[...], preferred_element_type=jnp.float32)
                    + wb_ref[...])


def _final(d, stats, g, be, W, wb):
    di, dn = W.shape
    return pl.pallas_call(
        _final_body,
        grid=(GRID,),
        in_specs=[_row_spec((N, di)), _full_spec((2, di)),
                  _full_spec((1, di)), _full_spec((1, di)),
                  _full_spec(W.shape), _full_spec((1, dn))],
        out_specs=_row_spec((N, dn)),
        out_shape=jax.ShapeDtypeStruct((N, dn), jnp.float32),
    )(d, stats, g, be, W, wb)


def _pack_vec(v, C):
    return jnp.concatenate(
        [jnp.tile(v[c * 16:(c + 1) * 16], 8) for c in range(C)]).reshape(1, -1)


def _wbig(W, C, Cn):
    I8 = jnp.eye(8, dtype=jnp.float32)
    return [jnp.kron(I8, W[c * 16:(c + 1) * 16, c2 * 16:(c2 + 1) * 16])
            for c in range(C) for c2 in range(Cn)]


def kernel(x, edge_index, W1, b1, g1, be1, W2, b2, g2, be2, W3, b3, g3, be3,
           W4, b4, g4, be4, cW1, cb1, cW2, cb2, cW3, cb3, cg1, cbe1,
           cg2, cbe2):
    pad = jnp.full((NW * KB * B - E,), N, jnp.int32)
    tail = jnp.full((NW, DEPTH, B), N, jnp.int32)

    def tile_idx(row):
        main = jnp.concatenate([row, pad]).reshape(NW, KB, B)
        return jnp.concatenate([main, tail], axis=1)

    src_t = tile_idx(edge_index[0])
    dst_t = tile_idx(edge_index[1])
    ones16 = jnp.ones((B, 16), jnp.float32)
    z16 = jnp.zeros((B, 16), jnp.float32)
    zF = jnp.zeros((B, F), jnp.float32)
    r2 = lambda v: v.reshape(1, -1)

    xp = jnp.pad(x, ((0, NA - N), (0, 0))).reshape(NP8, 336)
    W1bs = [jnp.kron(jnp.eye(8, dtype=jnp.float32),
                     W1[:, c * 16:(c + 1) * 16]) for c in range(C1)]
    degp = _deg_kernel(dst_t, ones16, z16).reshape(NC, NP8, 128)
    k0_out = _k0(xp, degp, W1bs)
    dinv, tabs = k0_out[0], list(k0_out[1:])

    def sc_run(C, tabs):
        flat = [t.reshape(NA, F) for t in tabs]
        parts = _scatter[C](src_t, dst_t, zF, *flat)
        parts = parts if isinstance(parts, (list, tuple)) else [parts]
        return [p.reshape(NC, NP8, 128) for p in parts]

    layer_params = [(b1, g1, be1, 4, W2, 8), (b2, g2, be2, 8, W3, 4),
                    (b3, g3, be3, 4, W4, 2)]
    for b_, g_, be_, C, Wn, Cn in layer_params:
        parts = sc_run(C, tabs)
        stats = _make_stats(C)(dinv, _pack_vec(b_, C), parts, tabs)
        tabs = _make_apply(C, Cn)(dinv, _pack_vec(b_, C), _pack_vec(g_, C),
                                  _pack_vec(be_, C), stats, _wbig(Wn, C, Cn),
                                  parts, tabs)
        tabs = list(tabs) if isinstance(tabs, (list, tuple)) else [tabs]

    parts = sc_run(2, tabs)
    stats4 = _make_stats(2)(dinv, _pack_vec(b4, 2), parts, tabs)
    a4 = _apply4(dinv, _pack_vec(b4, 2), _pack_vec(g4, 2), _pack_vec(be4, 2),
                 stats4, _wbig(cW1, 2, 2), _pack_vec(cb1, 2), parts, tabs)
    d1p, cs1lane = a4[:C4], a4[C4]
    d1 = jnp.concatenate([p.reshape(NA, 16) for p in d1p], axis=1)[:N]
    cs1 = cs1lane.reshape(2, C4, 8, 16).sum(axis=2).reshape(2, 32)
    d2, cs2 = _mlp(d1, cs1, r2(cg1), r2(cbe1), cW2, r2(cb2), 16)
    out = _final(d2, cs2, r2(cg2), r2(cbe2), cW3, r2(cb3))
    return out


# speedup vs baseline: 3.8991x; 1.0583x over previous
import functools

import jax
import jax.numpy as jnp
from jax import lax
from jax.experimental import pallas as pl
from jax.experimental.pallas import tpu as pltpu
from jax.experimental.pallas import tpu_sc as plsc

N = 50000
NA = 51200
E = 800000
NC, NS = 2, 16
NW = NC * NS
B = 128
KB = 200
DEPTH = 8
KBP = KB + DEPTH
EPAD = NW * KBP * B
F = 16
RB = 1000
GRID = N // RB
EPS = 1e-5

_MESH = plsc.VectorSubcoreMesh(
    core_axis_name="c", subcore_axis_name="s", num_cores=NC, num_subcores=NS)


def _zero_acc(zeros_v, acc, sid):
    base = sid * 3200
    for r in range(25):
        pltpu.sync_copy(zeros_v, acc.at[pl.ds(base + r * 128, 128)])


def _deg_body(dst_hbm, ones_hbm, z_hbm, out_hbm, dst_v, ones_v, zeros_v, acc):
    cc = lax.axis_index("c")
    sid = lax.axis_index("s")
    wid = cc * NS + sid
    pltpu.sync_copy(dst_hbm.at[wid], dst_v)
    pltpu.sync_copy(ones_hbm, ones_v)
    pltpu.sync_copy(z_hbm, zeros_v)
    _zero_acc(zeros_v, acc, sid)
    plsc.subcore_barrier()

    def body(j, car):
        pltpu.sync_copy(ones_v, acc.at[dst_v.at[j]], add=True)
        return car

    lax.fori_loop(0, KBP, body, 0)
    plsc.subcore_barrier()
    pltpu.sync_copy(acc.at[pl.ds(sid * 3200, 3200)],
                    out_hbm.at[cc, pl.ds(sid * 3200, 3200)])


_SC_PARAMS = pltpu.CompilerParams(use_tc_tiling_on_sc=False)

_deg_kernel = functools.partial(
    pl.kernel,
    out_type=jax.ShapeDtypeStruct((NC, NA, 16), jnp.float32),
    mesh=_MESH,
    compiler_params=_SC_PARAMS,
    scratch_types=[
        pltpu.VMEM((KBP, B), jnp.int32),
        pltpu.VMEM((B, 16), jnp.float32),
        pltpu.VMEM((B, 16), jnp.float32),
        pltpu.VMEM_SHARED((NA, 16), jnp.float32),
    ],
)(_deg_body)


def _make_scatter(C):
    def body(*refs):
        src_hbm, dst_hbm, z_hbm = refs[0], refs[1], refs[2]
        tabs = refs[3:3 + C]
        outs = refs[3 + C:3 + 2 * C]
        src8, dst8, zeros_v, buf, tab_s, acc, gsem, ssem = refs[3 + 2 * C:]
        cc = lax.axis_index("c")
        sid = lax.axis_index("s")
        wid = cc * NS + sid
        pltpu.sync_copy(z_hbm, zeros_v)
        sl = pl.ds(sid * 3200, 3200)
        bufs = [buf.at[pl.ds(k * B, B)] for k in range(4)]
        slabs = [(0, 72), (72, 72), (144, 56)]
        for c in range(C):
            pltpu.sync_copy(tabs[c].at[sl], tab_s.at[sl])
            _zero_acc(zeros_v, acc, sid)
            plsc.subcore_barrier()
            for off, n in slabs:
                pltpu.sync_copy(src_hbm.at[wid, pl.ds(off, n)],
                                src8.at[pl.ds(0, n)])
                pltpu.sync_copy(dst_hbm.at[wid, pl.ds(off, n)],
                                dst8.at[pl.ds(0, n)])
                for k in range(4):
                    pltpu.async_copy(tab_s.at[src8.at[k]], bufs[k], gsem)

                def bat(q, car):
                    for k in range(4):
                        pltpu.make_async_copy(
                            tab_s.at[src8.at[0]], bufs[k], gsem).wait()
                        pltpu.async_copy(bufs[k], acc.at[dst8.at[q + k]],
                                         ssem, add=True)
                    for k in range(4):
                        pltpu.make_async_copy(
                            bufs[k], acc.at[dst8.at[0]], ssem).wait()
                        pltpu.async_copy(
                            tab_s.at[src8.at[q + 4 + k]], bufs[k], gsem)
                    return car

                lax.fori_loop(0, (n - 4) // 4,
                              lambda it, car: bat(4 * it, car), 0)
                for k in range(4):
                    pltpu.make_async_copy(
                        tab_s.at[src8.at[0]], bufs[k], gsem).wait()
                    pltpu.sync_copy(bufs[k], acc.at[dst8.at[n - 4 + k]],
                                    add=True)
            plsc.subcore_barrier()
            pltpu.sync_copy(acc.at[sl], outs[c].at[cc, sl])
            plsc.subcore_barrier()

    return pl.kernel(
        body,
        out_type=[jax.ShapeDtypeStruct((NC, NA, F), jnp.float32)] * C,
        mesh=_MESH,
        compiler_params=_SC_PARAMS,
        scratch_types=[
            pltpu.VMEM((72, B), jnp.int32),
            pltpu.VMEM((72, B), jnp.int32),
            pltpu.VMEM((B, F), jnp.float32),
            pltpu.VMEM((4 * B, F), jnp.float32),
            pltpu.VMEM_SHARED((NA, F), jnp.float32),
            pltpu.VMEM_SHARED((NA, F), jnp.float32),
            pltpu.SemaphoreType.DMA,
            pltpu.SemaphoreType.DMA,
        ],
    )


_scatter = {C: _make_scatter(C) for C in (2, 4, 8)}


def _row_spec(shape):
    nd = len(shape)
    if nd == 2:
        return pl.BlockSpec((RB, shape[1]), lambda i: (i, 0))
    return pl.BlockSpec((shape[0], RB, shape[2]), lambda i: (0, i, 0))


def _full_spec(shape):
    return pl.BlockSpec(shape, lambda i: (0,) * len(shape))


C1 = 64 // F
RBP = 1024
PR = RBP // 8
NP8 = NA // 8
VROWS = N // 8


def _prow_spec(nd3=False, lanes=128):
    if nd3:
        return pl.BlockSpec((NC, PR, lanes), lambda i: (0, i, 0))
    return pl.BlockSpec((PR, lanes), lambda i: (i, 0))


def _full_spec(shape):
    return pl.BlockSpec(shape, lambda i: (0,) * len(shape))


def _row_spec(shape):
    return pl.BlockSpec((RB, shape[1]), lambda i: (i, 0))


def _bcast16(x):
    segs = [jnp.broadcast_to(x[:, k * 16:k * 16 + 1], (PR, 16))
            for k in range(8)]
    return jnp.concatenate(segs, axis=1)


def _fold16(s):
    return sum(s[:, k * 16:(k + 1) * 16] for k in range(8))


def _rowmask(i, x):
    gr = i * PR + lax.broadcasted_iota(jnp.int32, (PR, 1), 0)
    return jnp.where(gr < VROWS, x, 0.0)


def _k0_body(x_ref, dA_ref, W_refs_and_outs):
    pass


def _k0_body_impl(x_ref, dA_ref, Wb_refs, dinv_ref, t_refs):
    dsum = dA_ref[...][0] + dA_ref[...][1]
    deg = _bcast16(dsum) + 1.0
    dinv_p = lax.rsqrt(deg)
    dinv_ref[...] = dinv_p
    xp = x_ref[...]
    for c in range(C1):
        t_refs[c][...] = jnp.dot(
            xp, Wb_refs[c][...],
            preferred_element_type=jnp.float32) * dinv_p


def _k0(xp, degp, W1bs):
    def body(*refs):
        x_ref, dA_ref = refs[0], refs[1]
        Wb_refs = refs[2:2 + C1]
        dinv_ref = refs[2 + C1]
        t_refs = refs[3 + C1:]
        _k0_body_impl(x_ref, dA_ref, Wb_refs, dinv_ref, t_refs)

    return pl.pallas_call(
        body,
        grid=(GRID,),
        in_specs=[_prow_spec(lanes=336), _prow_spec(True)]
        + [_full_spec((336, 128))] * C1,
        out_specs=[_prow_spec()] * (1 + C1),
        out_shape=[jax.ShapeDtypeStruct((NP8, 128), jnp.float32)] * (1 + C1),
    )(xp, degp, *W1bs)


def _pre_chunks(dinv_p, b_ref, p_refs, t_refs):
    pres = []
    for c, (p, t) in enumerate(zip(p_refs, t_refs)):
        s = p[...][0] + p[...][1] + t[...]
        pres.append(s * dinv_p + b_ref[...][:, c * 128:(c + 1) * 128])
    return pres


def _make_stats(C):
    def body(*refs):
        dinv_ref, b_ref = refs[0], refs[1]
        p_refs = refs[2:2 + C]
        t_refs = refs[2 + C:2 + 2 * C]
        stats_ref, acc_ref = refs[2 + 2 * C], refs[3 + 2 * C]
        i = pl.program_id(0)
        pres = _pre_chunks(dinv_ref[...], b_ref, p_refs, t_refs)
        pre = pres[0] if C == 1 else jnp.concatenate(pres, axis=1)
        pre = _rowmask(i, pre)

        @pl.when(i == 0)
        def _():
            acc_ref[...] = jnp.zeros_like(acc_ref)

        acc_ref[...] += jnp.stack(
            [jnp.sum(pre, axis=0), jnp.sum(pre * pre, axis=0)])

        @pl.when(i == GRID - 1)
        def _():
            stats_ref[...] = acc_ref[...]

    def call(dinv, b, parts, tabs):
        C_ = len(tabs)
        return pl.pallas_call(
            body,
            grid=(GRID,),
            in_specs=[_prow_spec(), _full_spec((1, C_ * 128))]
            + [_prow_spec(True)] * C_ + [_prow_spec()] * C_,
            out_specs=_full_spec((2, C_ * 128)),
            out_shape=jax.ShapeDtypeStruct((2, C_ * 128), jnp.float32),
            scratch_shapes=[pltpu.VMEM((2, C_ * 128), jnp.float32)],
        )(dinv, b, *parts, *tabs)

    return call


def _bn_packed(pre_c, stats_ref, g_ref, be_ref, c):
    s = _fold16(stats_ref[...][:, c * 128:(c + 1) * 128])
    m16 = s[0:1, :] / N
    v16 = s[1:2, :] / N - m16 * m16
    r16 = lax.rsqrt(v16 + EPS)
    m_p = jnp.concatenate([m16] * 8, axis=1)
    r_p = jnp.concatenate([r16] * 8, axis=1)
    gseg = g_ref[...][:, c * 128:(c + 1) * 128]
    beseg = be_ref[...][:, c * 128:(c + 1) * 128]
    return (pre_c - m_p) * r_p * gseg + beseg


def _make_apply(C, C_next):
    def body(*refs):
        dinv_ref, b_ref, g_ref, be_ref, stats_ref = refs[:5]
        Wb_refs = refs[5:5 + C * C_next]
        p_refs = refs[5 + C * C_next:5 + C * C_next + C]
        t_refs = refs[5 + C * C_next + C:5 + C * C_next + 2 * C]
        o_refs = refs[5 + C * C_next + 2 * C:]
        dinv_p = dinv_ref[...]
        pres = _pre_chunks(dinv_p, b_ref, p_refs, t_refs)
        hs = [jnp.maximum(_bn_packed(pres[c], stats_ref, g_ref, be_ref, c),
                          0.0) for c in range(C)]
        for c2 in range(C_next):
            xw = hs[0] @ Wb_refs[c2][...]
            for c in range(1, C):
                xw += hs[c] @ Wb_refs[c * C_next + c2][...]
            o_refs[c2][...] = xw * dinv_p

    def call(dinv, b, g, be, stats, Wbs, parts, tabs):
        return pl.pallas_call(
            body,
            grid=(GRID,),
            in_specs=[_prow_spec(), _full_spec((1, C * 128)),
                      _full_spec((1, C * 128)), _full_spec((1, C * 128)),
                      _full_spec((2, C * 128))]
            + [_full_spec((128, 128))] * (C * C_next)
            + [_prow_spec(True)] * C + [_prow_spec()] * C,
            out_specs=[_prow_spec()] * C_next,
            out_shape=[jax.ShapeDtypeStruct((NP8, 128), jnp.float32)]
            * C_next,
        )(dinv, b, g, be, stats, *Wbs, *parts, *tabs)

    return call


C4 = 32 // F


def _apply4_body(*refs):
    dinv_ref, b_ref, g_ref, be_ref, stats_ref, cb_ref = refs[:6]
    Wb_refs = refs[6:6 + C4 * C4]
    p_refs = refs[6 + C4 * C4:6 + C4 * C4 + C4]
    t_refs = refs[6 + C4 * C4 + C4:6 + C4 * C4 + 2 * C4]
    d1_refs = refs[6 + C4 * C4 + 2 * C4:6 + C4 * C4 + 3 * C4]
    cstats_ref, acc_ref = refs[6 + C4 * C4 + 3 * C4:]
    i = pl.program_id(0)
    dinv_p = dinv_ref[...]
    pres = _pre_chunks(dinv_p, b_ref, p_refs, t_refs)
    hs = [jnp.maximum(_bn_packed(pres[c], stats_ref, g_ref, be_ref, c), 0.0)
          for c in range(C4)]
    d1m = []
    for c2 in range(C4):
        xw = hs[0] @ Wb_refs[c2][...]
        for c in range(1, C4):
            xw += hs[c] @ Wb_refs[c * C4 + c2][...]
        d1 = jnp.maximum(xw + cb_ref[...][:, c2 * 128:(c2 + 1) * 128], 0.0)
        d1_refs[c2][...] = d1
        d1m.append(_rowmask(i, d1))
    d1cat = jnp.concatenate(d1m, axis=1)

    @pl.when(i == 0)
    def _():
        acc_ref[...] = jnp.zeros_like(acc_ref)

    acc_ref[...] += jnp.stack(
        [jnp.sum(d1cat, axis=0), jnp.sum(d1cat * d1cat, axis=0)])

    @pl.when(i == GRID - 1)
    def _():
        cstats_ref[...] = acc_ref[...]


def _apply4(dinv, b, g, be, stats, cWbs, cb, parts, tabs):
    return pl.pallas_call(
        _apply4_body,
        grid=(GRID,),
        in_specs=[_prow_spec(), _full_spec((1, C4 * 128)),
                  _full_spec((1, C4 * 128)), _full_spec((1, C4 * 128)),
                  _full_spec((2, C4 * 128)), _full_spec((1, C4 * 128))]
        + [_full_spec((128, 128))] * (C4 * C4)
        + [_prow_spec(True)] * C4 + [_prow_spec()] * C4,
        out_specs=[_prow_spec()] * C4 + [_full_spec((2, C4 * 128))],
        out_shape=[jax.ShapeDtypeStruct((NP8, 128), jnp.float32)] * C4
        + [jax.ShapeDtypeStruct((2, C4 * 128), jnp.float32)],
        scratch_shapes=[pltpu.VMEM((2, C4 * 128), jnp.float32)],
    )(dinv, b, g, be, stats, cb, *cWbs, *parts, *tabs)

def _bn_apply(pre, stats_ref, g_ref, be_ref):
    m = stats_ref[...][0:1, :] / N
    v = stats_ref[...][1:2, :] / N - m * m
    rstd = lax.rsqrt(v + EPS)
    return (pre - m) * rstd * g_ref[...] + be_ref[...]


def _mlp_body(d_ref, stats_ref, g_ref, be_ref, W_ref, wb_ref,
              d2_ref, cstats_ref, acc_ref):
    i = pl.program_id(0)
    e = _bn_apply(d_ref[...], stats_ref, g_ref, be_ref)
    d2 = jnp.maximum(
        jnp.dot(e, W_ref[...], preferred_element_type=jnp.float32)
        + wb_ref[...], 0.0)
    d2_ref[...] = d2

    @pl.when(i == 0)
    def _():
        acc_ref[...] = jnp.zeros_like(acc_ref)

    acc_ref[...] += jnp.stack([jnp.sum(d2, axis=0), jnp.sum(d2 * d2, axis=0)])

    @pl.when(i == GRID - 1)
    def _():
        cstats_ref[...] = acc_ref[...]


def _mlp(d, stats, g, be, W, wb, dn):
    di = d.shape[1]
    return pl.pallas_call(
        _mlp_body,
        grid=(GRID,),
        in_specs=[_row_spec((N, di)), _full_spec((2, di)),
                  _full_spec((1, di)), _full_spec((1, di)),
                  _full_spec(W.shape), _full_spec((1, dn))],
        out_specs=[_row_spec((N, dn)), _full_spec((2, dn))],
        out_shape=[jax.ShapeDtypeStruct((N, dn), jnp.float32),
                   jax.ShapeDtypeStruct((2, dn), jnp.float32)],
        scratch_shapes=[pltpu.VMEM((2, dn), jnp.float32)],
    )(d, stats, g, be, W, wb)


def _final_body(d_ref, stats_ref, g_ref, be_ref, W_ref, wb_ref, out_ref):
    e = _bn_apply(d_ref[...], stats_ref, g_ref, be_ref)
    out_ref[...] = (jnp.dot(e, W_ref[...], preferred_element_type=jnp.float32)
                    + wb_ref[...])


def _final(d, stats, g, be, W, wb):
    di, dn = W.shape
    return pl.pallas_call(
        _final_body,
        grid=(GRID,),
        in_specs=[_row_spec((N, di)), _full_spec((2, di)),
                  _full_spec((1, di)), _full_spec((1, di)),
                  _full_spec(W.shape), _full_spec((1, dn))],
        out_specs=_row_spec((N, dn)),
        out_shape=jax.ShapeDtypeStruct((N, dn), jnp.float32),
    )(d, stats, g, be, W, wb)


def _pack_vec(v, C):
    return jnp.concatenate(
        [jnp.tile(v[c * 16:(c + 1) * 16], 8) for c in range(C)]).reshape(1, -1)


def _wbig(W, C, Cn):
    I8 = jnp.eye(8, dtype=jnp.float32)
    return [jnp.kron(I8, W[c * 16:(c + 1) * 16, c2 * 16:(c2 + 1) * 16])
            for c in range(C) for c2 in range(Cn)]


def kernel(x, edge_index, W1, b1, g1, be1, W2, b2, g2, be2, W3, b3, g3, be3,
           W4, b4, g4, be4, cW1, cb1, cW2, cb2, cW3, cb3, cg1, cbe1,
           cg2, cbe2):
    pad = jnp.full((NW * KB * B - E,), N, jnp.int32)
    tail = jnp.full((NW, DEPTH, B), N, jnp.int32)

    def tile_idx(row):
        main = jnp.concatenate([row, pad]).reshape(NW, KB, B)
        return jnp.concatenate([main, tail], axis=1)

    src_t = tile_idx(edge_index[0])
    dst_t = tile_idx(edge_index[1])
    ones16 = jnp.ones((B, 16), jnp.float32)
    z16 = jnp.zeros((B, 16), jnp.float32)
    zF = jnp.zeros((B, F), jnp.float32)
    r2 = lambda v: v.reshape(1, -1)

    xp = jnp.pad(x, ((0, NA - N), (0, 0))).reshape(NP8, 336)
    W1bs = [jnp.kron(jnp.eye(8, dtype=jnp.float32),
                     W1[:, c * 16:(c + 1) * 16]) for c in range(C1)]
    degp = _deg_kernel(dst_t, ones16, z16).reshape(NC, NP8, 128)
    k0_out = _k0(xp, degp, W1bs)
    dinv, tabs = k0_out[0], list(k0_out[1:])

    def sc_run(C, tabs):
        flat = [t.reshape(NA, F) for t in tabs]
        parts = _scatter[C](src_t, dst_t, zF, *flat)
        parts = parts if isinstance(parts, (list, tuple)) else [parts]
        return [p.reshape(NC, NP8, 128) for p in parts]

    layer_params = [(b1, g1, be1, 4, W2, 8), (b2, g2, be2, 8, W3, 4),
                    (b3, g3, be3, 4, W4, 2)]
    for b_, g_, be_, C, Wn, Cn in layer_params:
        parts = sc_run(C, tabs)
        stats = _make_stats(C)(dinv, _pack_vec(b_, C), parts, tabs)
        tabs = _make_apply(C, Cn)(dinv, _pack_vec(b_, C), _pack_vec(g_, C),
                                  _pack_vec(be_, C), stats, _wbig(Wn, C, Cn),
                                  parts, tabs)
        tabs = list(tabs) if isinstance(tabs, (list, tuple)) else [tabs]

    parts = sc_run(2, tabs)
    stats4 = _make_stats(2)(dinv, _pack_vec(b4, 2), parts, tabs)
    a4 = _apply4(dinv, _pack_vec(b4, 2), _pack_vec(g4, 2), _pack_vec(be4, 2),
                 stats4, _wbig(cW1, 2, 2), _pack_vec(cb1, 2), parts, tabs)
    d1p, cs1lane = a4[:C4], a4[C4]
    d1 = jnp.concatenate([p.reshape(NA, 16) for p in d1p], axis=1)[:N]
    cs1 = cs1lane.reshape(2, C4, 8, 16).sum(axis=2).reshape(2, 32)
    d2, cs2 = _mlp(d1, cs1, r2(cg1), r2(cbe1), cW2, r2(cb2), 16)
    out = _final(d2, cs2, r2(cg2), r2(cbe2), cW3, r2(cb3))
    return out


# speedup vs baseline: 4.0196x; 1.0309x over previous
import functools

import jax
import jax.numpy as jnp
from jax import lax
from jax.experimental import pallas as pl
from jax.experimental.pallas import tpu as pltpu
from jax.experimental.pallas import tpu_sc as plsc

N = 50000
NA = 51200
E = 800000
NC, NS = 2, 16
NW = NC * NS
B = 128
KB = 200
DEPTH = 8
KBP = KB + DEPTH
EPAD = NW * KBP * B
F = 16
RB = 1000
GRID = N // RB
EPS = 1e-5

_MESH = plsc.VectorSubcoreMesh(
    core_axis_name="c", subcore_axis_name="s", num_cores=NC, num_subcores=NS)


def _zero_acc(zeros_v, acc, sid):
    base = sid * 3200
    for r in range(25):
        pltpu.sync_copy(zeros_v, acc.at[pl.ds(base + r * 128, 128)])


def _deg_body(dst_hbm, ones_hbm, z_hbm, out_hbm, dst_v, ones_v, zeros_v, acc):
    cc = lax.axis_index("c")
    sid = lax.axis_index("s")
    wid = cc * NS + sid
    pltpu.sync_copy(dst_hbm.at[wid], dst_v)
    pltpu.sync_copy(ones_hbm, ones_v)
    pltpu.sync_copy(z_hbm, zeros_v)
    _zero_acc(zeros_v, acc, sid)
    plsc.subcore_barrier()

    def body(j, car):
        pltpu.sync_copy(ones_v, acc.at[dst_v.at[j]], add=True)
        return car

    lax.fori_loop(0, KBP, body, 0)
    plsc.subcore_barrier()
    pltpu.sync_copy(acc.at[pl.ds(sid * 3200, 3200)],
                    out_hbm.at[cc, pl.ds(sid * 3200, 3200)])


_SC_PARAMS = pltpu.CompilerParams(use_tc_tiling_on_sc=False)

_deg_kernel = functools.partial(
    pl.kernel,
    out_type=jax.ShapeDtypeStruct((NC, NA, 16), jnp.float32),
    mesh=_MESH,
    compiler_params=_SC_PARAMS,
    scratch_types=[
        pltpu.VMEM((KBP, B), jnp.int32),
        pltpu.VMEM((B, 16), jnp.float32),
        pltpu.VMEM((B, 16), jnp.float32),
        pltpu.VMEM_SHARED((NA, 16), jnp.float32),
    ],
)(_deg_body)


def _make_scatter(C):
    def body(*refs):
        src_hbm, dst_hbm, z_hbm = refs[0], refs[1], refs[2]
        tabs = refs[3:3 + C]
        outs = refs[3 + C:3 + 2 * C]
        src8, dst8, zeros_v, buf, tab_s, acc, gsem, ssem = refs[3 + 2 * C:]
        cc = lax.axis_index("c")
        sid = lax.axis_index("s")
        wid = cc * NS + sid
        pltpu.sync_copy(z_hbm, zeros_v)
        sl = pl.ds(sid * 3200, 3200)
        bufs = [buf.at[pl.ds(k * B, B)] for k in range(4)]
        slabs = [(0, 72), (72, 72), (144, 56)]
        pltpu.sync_copy(tabs[0].at[sl], tab_s.at[sl])
        _zero_acc(zeros_v, acc, sid)
        plsc.subcore_barrier()
        for c in range(C):
            for off, n in slabs:
                pltpu.sync_copy(src_hbm.at[wid, pl.ds(off, n)],
                                src8.at[pl.ds(0, n)])
                pltpu.sync_copy(dst_hbm.at[wid, pl.ds(off, n)],
                                dst8.at[pl.ds(0, n)])
                for k in range(4):
                    pltpu.async_copy(tab_s.at[src8.at[k]], bufs[k], gsem)

                def bat(q, car):
                    for k in range(4):
                        pltpu.make_async_copy(
                            tab_s.at[src8.at[0]], bufs[k], gsem).wait()
                        pltpu.async_copy(bufs[k], acc.at[dst8.at[q + k]],
                                         ssem, add=True)
                    for k in range(4):
                        pltpu.make_async_copy(
                            bufs[k], acc.at[dst8.at[0]], ssem).wait()
                        pltpu.async_copy(
                            tab_s.at[src8.at[q + 4 + k]], bufs[k], gsem)
                    return car

                lax.fori_loop(0, (n - 4) // 4,
                              lambda it, car: bat(4 * it, car), 0)
                for k in range(4):
                    pltpu.make_async_copy(
                        tab_s.at[src8.at[0]], bufs[k], gsem).wait()
                    pltpu.sync_copy(bufs[k], acc.at[dst8.at[n - 4 + k]],
                                    add=True)
            plsc.subcore_barrier()
            pltpu.async_copy(acc.at[sl], outs[c].at[cc, sl], gsem)
            if c + 1 < C:
                pltpu.sync_copy(tabs[c + 1].at[sl], tab_s.at[sl])
            pltpu.make_async_copy(acc.at[sl], outs[c].at[cc, sl],
                                  gsem).wait()
            if c + 1 < C:
                _zero_acc(zeros_v, acc, sid)
                plsc.subcore_barrier()

    return pl.kernel(
        body,
        out_type=[jax.ShapeDtypeStruct((NC, NA, F), jnp.float32)] * C,
        mesh=_MESH,
        compiler_params=_SC_PARAMS,
        scratch_types=[
            pltpu.VMEM((72, B), jnp.int32),
            pltpu.VMEM((72, B), jnp.int32),
            pltpu.VMEM((B, F), jnp.float32),
            pltpu.VMEM((4 * B, F), jnp.float32),
            pltpu.VMEM_SHARED((NA, F), jnp.float32),
            pltpu.VMEM_SHARED((NA, F), jnp.float32),
            pltpu.SemaphoreType.DMA,
            pltpu.SemaphoreType.DMA,
        ],
    )


_scatter = {C: _make_scatter(C) for C in (2, 4, 8)}


def _row_spec(shape):
    nd = len(shape)
    if nd == 2:
        return pl.BlockSpec((RB, shape[1]), lambda i: (i, 0))
    return pl.BlockSpec((shape[0], RB, shape[2]), lambda i: (0, i, 0))


def _full_spec(shape):
    return pl.BlockSpec(shape, lambda i: (0,) * len(shape))


C1 = 64 // F
RBP = 1024
PR = RBP // 8
NP8 = NA // 8
VROWS = N // 8


def _prow_spec(nd3=False, lanes=128):
    if nd3:
        return pl.BlockSpec((NC, PR, lanes), lambda i: (0, i, 0))
    return pl.BlockSpec((PR, lanes), lambda i: (i, 0))


def _full_spec(shape):
    return pl.BlockSpec(shape, lambda i: (0,) * len(shape))


def _row_spec(shape):
    return pl.BlockSpec((RB, shape[1]), lambda i: (i, 0))


def _bcast16(x):
    segs = [jnp.broadcast_to(x[:, k * 16:k * 16 + 1], (PR, 16))
            for k in range(8)]
    return jnp.concatenate(segs, axis=1)


def _fold16(s):
    return sum(s[:, k * 16:(k + 1) * 16] for k in range(8))


def _rowmask(i, x):
    gr = i * PR + lax.broadcasted_iota(jnp.int32, (PR, 1), 0)
    return jnp.where(gr < VROWS, x, 0.0)


def _k0_body(x_ref, dA_ref, W_refs_and_outs):
    pass


def _k0_body_impl(x_ref, dA_ref, Wb_refs, dinv_ref, t_refs):
    dsum = dA_ref[...][0] + dA_ref[...][1]
    deg = _bcast16(dsum) + 1.0
    dinv_p = lax.rsqrt(deg)
    dinv_ref[...] = dinv_p
    xp = x_ref[...]
    for c in range(C1):
        t_refs[c][...] = jnp.dot(
            xp, Wb_refs[c][...],
            preferred_element_type=jnp.float32) * dinv_p


def _k0(xp, degp, W1bs):
    def body(*refs):
        x_ref, dA_ref = refs[0], refs[1]
        Wb_refs = refs[2:2 + C1]
        dinv_ref = refs[2 + C1]
        t_refs = refs[3 + C1:]
        _k0_body_impl(x_ref, dA_ref, Wb_refs, dinv_ref, t_refs)

    return pl.pallas_call(
        body,
        grid=(GRID,),
        in_specs=[_prow_spec(lanes=336), _prow_spec(True)]
        + [_full_spec((336, 128))] * C1,
        out_specs=[_prow_spec()] * (1 + C1),
        out_shape=[jax.ShapeDtypeStruct((NP8, 128), jnp.float32)] * (1 + C1),
    )(xp, degp, *W1bs)


def _pre_chunks(dinv_p, b_ref, p_refs, t_refs):
    pres = []
    for c, (p, t) in enumerate(zip(p_refs, t_refs)):
        s = p[...][0] + p[...][1] + t[...]
        pres.append(s * dinv_p + b_ref[...][:, c * 128:(c + 1) * 128])
    return pres


def _make_stats(C):
    def body(*refs):
        dinv_ref, b_ref = refs[0], refs[1]
        p_refs = refs[2:2 + C]
        t_refs = refs[2 + C:2 + 2 * C]
        stats_ref, acc_ref = refs[2 + 2 * C], refs[3 + 2 * C]
        i = pl.program_id(0)
        pres = _pre_chunks(dinv_ref[...], b_ref, p_refs, t_refs)
        pre = pres[0] if C == 1 else jnp.concatenate(pres, axis=1)
        pre = _rowmask(i, pre)

        @pl.when(i == 0)
        def _():
            acc_ref[...] = jnp.zeros_like(acc_ref)

        acc_ref[...] += jnp.stack(
            [jnp.sum(pre, axis=0), jnp.sum(pre * pre, axis=0)])

        @pl.when(i == GRID - 1)
        def _():
            stats_ref[...] = acc_ref[...]

    def call(dinv, b, parts, tabs):
        C_ = len(tabs)
        return pl.pallas_call(
            body,
            grid=(GRID,),
            in_specs=[_prow_spec(), _full_spec((1, C_ * 128))]
            + [_prow_spec(True)] * C_ + [_prow_spec()] * C_,
            out_specs=_full_spec((2, C_ * 128)),
            out_shape=jax.ShapeDtypeStruct((2, C_ * 128), jnp.float32),
            scratch_shapes=[pltpu.VMEM((2, C_ * 128), jnp.float32)],
        )(dinv, b, *parts, *tabs)

    return call


def _bn_packed(pre_c, stats_ref, g_ref, be_ref, c):
    s = _fold16(stats_ref[...][:, c * 128:(c + 1) * 128])
    m16 = s[0:1, :] / N
    v16 = s[1:2, :] / N - m16 * m16
    r16 = lax.rsqrt(v16 + EPS)
    m_p = jnp.concatenate([m16] * 8, axis=1)
    r_p = jnp.concatenate([r16] * 8, axis=1)
    gseg = g_ref[...][:, c * 128:(c + 1) * 128]
    beseg = be_ref[...][:, c * 128:(c + 1) * 128]
    return (pre_c - m_p) * r_p * gseg + beseg


def _make_apply(C, C_next):
    def body(*refs):
        dinv_ref, b_ref, g_ref, be_ref, stats_ref = refs[:5]
        Wb_refs = refs[5:5 + C * C_next]
        p_refs = refs[5 + C * C_next:5 + C * C_next + C]
        t_refs = refs[5 + C * C_next + C:5 + C * C_next + 2 * C]
        o_refs = refs[5 + C * C_next + 2 * C:]
        dinv_p = dinv_ref[...]
        pres = _pre_chunks(dinv_p, b_ref, p_refs, t_refs)
        hs = [jnp.maximum(_bn_packed(pres[c], stats_ref, g_ref, be_ref, c),
                          0.0) for c in range(C)]
        for c2 in range(C_next):
            xw = hs[0] @ Wb_refs[c2][...]
            for c in range(1, C):
                xw += hs[c] @ Wb_refs[c * C_next + c2][...]
            o_refs[c2][...] = xw * dinv_p

    def call(dinv, b, g, be, stats, Wbs, parts, tabs):
        return pl.pallas_call(
            body,
            grid=(GRID,),
            in_specs=[_prow_spec(), _full_spec((1, C * 128)),
                      _full_spec((1, C * 128)), _full_spec((1, C * 128)),
                      _full_spec((2, C * 128))]
            + [_full_spec((128, 128))] * (C * C_next)
            + [_prow_spec(True)] * C + [_prow_spec()] * C,
            out_specs=[_prow_spec()] * C_next,
            out_shape=[jax.ShapeDtypeStruct((NP8, 128), jnp.float32)]
            * C_next,
        )(dinv, b, g, be, stats, *Wbs, *parts, *tabs)

    return call


C4 = 32 // F


def _apply4_body(*refs):
    dinv_ref, b_ref, g_ref, be_ref, stats_ref, cb_ref = refs[:6]
    Wb_refs = refs[6:6 + C4 * C4]
    p_refs = refs[6 + C4 * C4:6 + C4 * C4 + C4]
    t_refs = refs[6 + C4 * C4 + C4:6 + C4 * C4 + 2 * C4]
    d1_refs = refs[6 + C4 * C4 + 2 * C4:6 + C4 * C4 + 3 * C4]
    cstats_ref, acc_ref = refs[6 + C4 * C4 + 3 * C4:]
    i = pl.program_id(0)
    dinv_p = dinv_ref[...]
    pres = _pre_chunks(dinv_p, b_ref, p_refs, t_refs)
    hs = [jnp.maximum(_bn_packed(pres[c], stats_ref, g_ref, be_ref, c), 0.0)
          for c in range(C4)]
    d1m = []
    for c2 in range(C4):
        xw = hs[0] @ Wb_refs[c2][...]
        for c in range(1, C4):
            xw += hs[c] @ Wb_refs[c * C4 + c2][...]
        d1 = jnp.maximum(xw + cb_ref[...][:, c2 * 128:(c2 + 1) * 128], 0.0)
        d1_refs[c2][...] = d1
        d1m.append(_rowmask(i, d1))
    d1cat = jnp.concatenate(d1m, axis=1)

    @pl.when(i == 0)
    def _():
        acc_ref[...] = jnp.zeros_like(acc_ref)

    acc_ref[...] += jnp.stack(
        [jnp.sum(d1cat, axis=0), jnp.sum(d1cat * d1cat, axis=0)])

    @pl.when(i == GRID - 1)
    def _():
        cstats_ref[...] = acc_ref[...]


def _apply4(dinv, b, g, be, stats, cWbs, cb, parts, tabs):
    return pl.pallas_call(
        _apply4_body,
        grid=(GRID,),
        in_specs=[_prow_spec(), _full_spec((1, C4 * 128)),
                  _full_spec((1, C4 * 128)), _full_spec((1, C4 * 128)),
                  _full_spec((2, C4 * 128)), _full_spec((1, C4 * 128))]
        + [_full_spec((128, 128))] * (C4 * C4)
        + [_prow_spec(True)] * C4 + [_prow_spec()] * C4,
        out_specs=[_prow_spec()] * C4 + [_full_spec((2, C4 * 128))],
        out_shape=[jax.ShapeDtypeStruct((NP8, 128), jnp.float32)] * C4
        + [jax.ShapeDtypeStruct((2, C4 * 128), jnp.float32)],
        scratch_shapes=[pltpu.VMEM((2, C4 * 128), jnp.float32)],
    )(dinv, b, g, be, stats, cb, *cWbs, *parts, *tabs)

def _bn_apply(pre, stats_ref, g_ref, be_ref):
    m = stats_ref[...][0:1, :] / N
    v = stats_ref[...][1:2, :] / N - m * m
    rstd = lax.rsqrt(v + EPS)
    return (pre - m) * rstd * g_ref[...] + be_ref[...]


def _mlp_body(d_ref, stats_ref, g_ref, be_ref, W_ref, wb_ref,
              d2_ref, cstats_ref, acc_ref):
    i = pl.program_id(0)
    e = _bn_apply(d_ref[...], stats_ref, g_ref, be_ref)
    d2 = jnp.maximum(
        jnp.dot(e, W_ref[...], preferred_element_type=jnp.float32)
        + wb_ref[...], 0.0)
    d2_ref[...] = d2

    @pl.when(i == 0)
    def _():
        acc_ref[...] = jnp.zeros_like(acc_ref)

    acc_ref[...] += jnp.stack([jnp.sum(d2, axis=0), jnp.sum(d2 * d2, axis=0)])

    @pl.when(i == GRID - 1)
    def _():
        cstats_ref[...] = acc_ref[...]


def _mlp(d, stats, g, be, W, wb, dn):
    di = d.shape[1]
    return pl.pallas_call(
        _mlp_body,
        grid=(GRID,),
        in_specs=[_row_spec((N, di)), _full_spec((2, di)),
                  _full_spec((1, di)), _full_spec((1, di)),
                  _full_spec(W.shape), _full_spec((1, dn))],
        out_specs=[_row_spec((N, dn)), _full_spec((2, dn))],
        out_shape=[jax.ShapeDtypeStruct((N, dn), jnp.float32),
                   jax.ShapeDtypeStruct((2, dn), jnp.float32)],
        scratch_shapes=[pltpu.VMEM((2, dn), jnp.float32)],
    )(d, stats, g, be, W, wb)


def _final_body(d_ref, stats_ref, g_ref, be_ref, W_ref, wb_ref, out_ref):
    e = _bn_apply(d_ref[...], stats_ref, g_ref, be_ref)
    out_ref[...] = (jnp.dot(e, W_ref[...], preferred_element_type=jnp.float32)
                    + wb_ref[...])


def _final(d, stats, g, be, W, wb):
    di, dn = W.shape
    return pl.pallas_call(
        _final_body,
        grid=(GRID,),
        in_specs=[_row_spec((N, di)), _full_spec((2, di)),
                  _full_spec((1, di)), _full_spec((1, di)),
                  _full_spec(W.shape), _full_spec((1, dn))],
        out_specs=_row_spec((N, dn)),
        out_shape=jax.ShapeDtypeStruct((N, dn), jnp.float32),
    )(d, stats, g, be, W, wb)


def _pack_vec(v, C):
    return jnp.concatenate(
        [jnp.tile(v[c * 16:(c + 1) * 16], 8) for c in range(C)]).reshape(1, -1)


def _wbig(W, C, Cn):
    I8 = jnp.eye(8, dtype=jnp.float32)
    return [jnp.kron(I8, W[c * 16:(c + 1) * 16, c2 * 16:(c2 + 1) * 16])
            for c in range(C) for c2 in range(Cn)]


def kernel(x, edge_index, W1, b1, g1, be1, W2, b2, g2, be2, W3, b3, g3, be3,
           W4, b4, g4, be4, cW1, cb1, cW2, cb2, cW3, cb3, cg1, cbe1,
           cg2, cbe2):
    pad = jnp.full((NW * KB * B - E,), N, jnp.int32)
    tail = jnp.full((NW, DEPTH, B), N, jnp.int32)

    def tile_idx(row):
        main = jnp.concatenate([row, pad]).reshape(NW, KB, B)
        return jnp.concatenate([main, tail], axis=1)

    src_t = tile_idx(edge_index[0])
    dst_t = tile_idx(edge_index[1])
    ones16 = jnp.ones((B, 16), jnp.float32)
    z16 = jnp.zeros((B, 16), jnp.float32)
    zF = jnp.zeros((B, F), jnp.float32)
    r2 = lambda v: v.reshape(1, -1)

    xp = jnp.pad(x, ((0, NA - N), (0, 0))).reshape(NP8, 336)
    W1bs = [jnp.kron(jnp.eye(8, dtype=jnp.float32),
                     W1[:, c * 16:(c + 1) * 16]) for c in range(C1)]
    degp = _deg_kernel(dst_t, ones16, z16).reshape(NC, NP8, 128)
    k0_out = _k0(xp, degp, W1bs)
    dinv, tabs = k0_out[0], list(k0_out[1:])

    def sc_run(C, tabs):
        flat = [t.reshape(NA, F) for t in tabs]
        parts = _scatter[C](src_t, dst_t, zF, *flat)
        parts = parts if isinstance(parts, (list, tuple)) else [parts]
        return [p.reshape(NC, NP8, 128) for p in parts]

    layer_params = [(b1, g1, be1, 4, W2, 8), (b2, g2, be2, 8, W3, 4),
                    (b3, g3, be3, 4, W4, 2)]
    for b_, g_, be_, C, Wn, Cn in layer_params:
        parts = sc_run(C, tabs)
        stats = _make_stats(C)(dinv, _pack_vec(b_, C), parts, tabs)
        tabs = _make_apply(C, Cn)(dinv, _pack_vec(b_, C), _pack_vec(g_, C),
                                  _pack_vec(be_, C), stats, _wbig(Wn, C, Cn),
                                  parts, tabs)
        tabs = list(tabs) if isinstance(tabs, (list, tuple)) else [tabs]

    parts = sc_run(2, tabs)
    stats4 = _make_stats(2)(dinv, _pack_vec(b4, 2), parts, tabs)
    a4 = _apply4(dinv, _pack_vec(b4, 2), _pack_vec(g4, 2), _pack_vec(be4, 2),
                 stats4, _wbig(cW1, 2, 2), _pack_vec(cb1, 2), parts, tabs)
    d1p, cs1lane = a4[:C4], a4[C4]
    d1 = jnp.concatenate([p.reshape(NA, 16) for p in d1p], axis=1)[:N]
    cs1 = cs1lane.reshape(2, C4, 8, 16).sum(axis=2).reshape(2, 32)
    d2, cs2 = _mlp(d1, cs1, r2(cg1), r2(cbe1), cW2, r2(cb2), 16)
    out = _final(d2, cs2, r2(cg2), r2(cbe2), cW3, r2(cb3))
    return out
